# baseline (speedup 1.0000x reference)
import functools

import jax
import jax.numpy as jnp
from jax.experimental import pallas as pl
from jax.experimental.pallas import tpu as pltpu

_BN_EPS = 1e-5


def _rup(x, m):
    return ((x + m - 1) // m) * m


def _cdiv(a, b):
    return -(-a // b)


def _up_kernel(x_ref, w_ref, b_ref, o_ref):
    acc = jnp.dot(x_ref[...], w_ref[...], preferred_element_type=jnp.float32)
    o_ref[...] = (acc + b_ref[...]).astype(o_ref.dtype)


def _up_matmul(x, w, b):
    M, K = x.shape
    Ncol = w.shape[1]
    bm = min(2048, _rup(M, 8))
    nb = _cdiv(M, bm)
    rows = nb * bm
    if rows != M:
        x = jnp.pad(x, ((0, rows - M), (0, 0)))
    out = pl.pallas_call(
        _up_kernel,
        out_shape=jax.ShapeDtypeStruct((rows, Ncol), jnp.bfloat16),
        grid=(nb,),
        in_specs=[
            pl.BlockSpec((bm, K), lambda i: (i, 0)),
            pl.BlockSpec((K, Ncol), lambda i: (0, 0)),
            pl.BlockSpec((1, Ncol), lambda i: (0, 0)),
        ],
        out_specs=pl.BlockSpec((bm, Ncol), lambda i: (i, 0)),
        compiler_params=pltpu.CompilerParams(
            dimension_semantics=("parallel",),
            vmem_limit_bytes=96 * 1024 * 1024),
    )(x, w, b)
    return out[:M]


def _dconv_kernel(y_ref, yh1_ref, yh2_ref, x2_ref, xh1_ref, xh2_ref,
                  w1_ref, s1_ref, w2_ref, s2_ref, o_ref, win_ref, h_ref,
                  *, bm, hp, Ws, Hs, H2, W2, M1, Kc, Co):
    ktot = 2 * Kc
    win_ref[0:bm, 0:Kc] = y_ref[...]
    win_ref[bm:bm + hp, 0:Kc] = yh1_ref[...]
    win_ref[bm + hp:bm + 2 * hp, 0:Kc] = yh2_ref[...]
    win_ref[0:bm, Kc:ktot] = x2_ref[...]
    win_ref[bm:bm + hp, Kc:ktot] = xh1_ref[...]
    win_ref[bm + hp:bm + 2 * hp, Kc:ktot] = xh2_ref[...]

    offs = tuple(dy * Ws + dx for dy in range(3) for dx in range(3))
    Mh = bm + hp

    acc = jnp.dot(win_ref[offs[0]:offs[0] + Mh, :], w1_ref[0:ktot, :],
                  preferred_element_type=jnp.float32)
    for t in range(1, 9):
        acc = acc + jnp.dot(win_ref[offs[t]:offs[t] + Mh, :],
                            w1_ref[t * ktot:(t + 1) * ktot, :],
                            preferred_element_type=jnp.float32)

    idx = pl.program_id(0) * bm + jax.lax.broadcasted_iota(jnp.int32, (Mh, 1), 0)
    r = idx % (Hs * Ws)
    yy = r // Ws
    xx = r - yy * Ws
    keep = ((yy >= 1) & (yy <= H2) & (xx >= 1) & (xx <= W2) & (idx < M1))
    hval = jnp.maximum(acc + s1_ref[...], 0.0)
    h_ref[...] = jnp.where(keep, hval, 0.0).astype(h_ref.dtype)

    acc2 = jnp.dot(h_ref[offs[0]:offs[0] + bm, :], w2_ref[0:Co, :],
                   preferred_element_type=jnp.float32)
    for t in range(1, 9):
        acc2 = acc2 + jnp.dot(h_ref[offs[t]:offs[t] + bm, :],
                              w2_ref[t * Co:(t + 1) * Co, :],
                              preferred_element_type=jnp.float32)
    o_ref[...] = jnp.maximum(acc2 + s2_ref[...], 0.0).astype(o_ref.dtype)


def _double_conv(y_rows, x2_rows, w1_9, s1, w2_9, s2,
                 *, bm, hp, nb, Ws, Hs, H2, W2, M1, out_dtype):
    rows = nb * bm
    Kc = int(y_rows.shape[1])
    Co = int(w2_9.shape[1])
    blk = bm // hp
    tot = nb * blk

    def hmap(d):
        return lambda i, d=d: ((i * blk + blk + d) % tot, 0)

    in_specs = [
        pl.BlockSpec((bm, Kc), lambda i: (i, 0)),
        pl.BlockSpec((hp, Kc), hmap(0)),
        pl.BlockSpec((hp, Kc), hmap(1)),
        pl.BlockSpec((bm, Kc), lambda i: (i, 0)),
        pl.BlockSpec((hp, Kc), hmap(0)),
        pl.BlockSpec((hp, Kc), hmap(1)),
        pl.BlockSpec(w1_9.shape, lambda i: (0, 0)),
        pl.BlockSpec((1, Co), lambda i: (0, 0)),
        pl.BlockSpec(w2_9.shape, lambda i: (0, 0)),
        pl.BlockSpec((1, Co), lambda i: (0, 0)),
    ]
    body = functools.partial(_dconv_kernel, bm=bm, hp=hp, Ws=Ws, Hs=Hs,
                             H2=H2, W2=W2, M1=M1, Kc=Kc, Co=Co)
    return pl.pallas_call(
        body,
        out_shape=jax.ShapeDtypeStruct((rows, Co), out_dtype),
        grid=(nb,),
        in_specs=in_specs,
        out_specs=pl.BlockSpec((bm, Co), lambda i: (i, 0)),
        scratch_shapes=[pltpu.VMEM((bm + 2 * hp, 2 * Kc), jnp.bfloat16),
                        pltpu.VMEM((bm + hp, Co), jnp.bfloat16)],
        compiler_params=pltpu.CompilerParams(
            dimension_semantics=("parallel",),
            vmem_limit_bytes=96 * 1024 * 1024),
    )(y_rows, y_rows, y_rows, x2_rows, x2_rows, x2_rows, w1_9, s1, w2_9, s2)


def kernel(x1_nchw, x2_nchw, w_up, b_up, w1, b1, g1, be1, w2, b2, g2, be2):
    x1 = jnp.transpose(x1_nchw, (0, 2, 3, 1)).astype(jnp.bfloat16)
    x2 = jnp.transpose(x2_nchw, (0, 2, 3, 1)).astype(jnp.bfloat16)
    N, H, W, C1 = x1.shape
    _, H2, W2, Ch = x2.shape
    Co = int(w1.shape[0])

    wt = jnp.transpose(w_up, (0, 2, 3, 1)).reshape(C1, 4 * Ch).astype(jnp.bfloat16)
    bu = jnp.broadcast_to(b_up[None, None, :], (2, 2, Ch)).reshape(1, 4 * Ch)
    y = _up_matmul(x1.reshape(N * H * W, C1), wt, bu)
    y = (y.reshape(N, H, W, 2, 2, Ch)
          .transpose(0, 1, 3, 2, 4, 5)
          .reshape(N, 2 * H, 2 * W, Ch))

    diffY = H2 - 2 * H
    diffX = W2 - 2 * W
    Hs = H2 + 4
    Ws = _rup(W2 + 4, 8)
    top, left = 2 + diffY // 2, 2 + diffX // 2
    y_slab = jnp.pad(y, ((0, 0), (top, Hs - top - 2 * H),
                         (left, Ws - left - 2 * W), (0, 0)))
    x2_slab = jnp.pad(x2, ((0, 0), (2, Hs - 2 - H2), (2, Ws - 2 - W2), (0, 0)))

    M1 = N * Hs * Ws
    hp = _rup(2 * Ws + 2, 8)
    blk = 16
    while blk > 1 and _cdiv(M1, blk * hp) < 4:
        blk //= 2
    bm = blk * hp
    nb = _cdiv(M1, bm)
    rows = nb * bm

    def rowsify(s):
        f = s.reshape(M1, s.shape[-1])
        return jnp.pad(f, ((0, rows - M1), (0, 0))) if rows != M1 else f

    scale1 = g1 / jnp.sqrt(1.0 + _BN_EPS)
    w1_9 = (jnp.transpose(w1, (2, 3, 1, 0)) * scale1).reshape(9 * C1, Co)
    w1_9 = w1_9.astype(jnp.bfloat16)
    s1 = (b1 * scale1 + be1).reshape(1, Co)
    scale2 = g2 / jnp.sqrt(1.0 + _BN_EPS)
    w2_9 = (jnp.transpose(w2, (2, 3, 1, 0)) * scale2).reshape(9 * Co, Co)
    w2_9 = w2_9.astype(jnp.bfloat16)
    s2 = (b2 * scale2 + be2).reshape(1, Co)

    o = _double_conv(rowsify(y_slab), rowsify(x2_slab), w1_9, s1, w2_9, s2,
                     bm=bm, hp=hp, nb=nb, Ws=Ws, Hs=Hs, H2=H2, W2=W2, M1=M1,
                     out_dtype=jnp.float32)
    out = o[:M1].reshape(N, Hs, Ws, Co)[:, :H2, :W2, :]
    return jnp.transpose(out, (0, 3, 1, 2))


# speedup vs baseline: 1.4270x; 1.4270x over previous
import functools

import jax
import jax.numpy as jnp
from jax.experimental import pallas as pl
from jax.experimental.pallas import tpu as pltpu

_BN_EPS = 1e-5


def _rup(x, m):
    return ((x + m - 1) // m) * m


def _up_block_kernel(x1_ref, x2_ref, wup_ref, bu_ref, w1_ref, s1_ref,
                     w2_ref, s2_ref, o_ref, win_ref, h_ref, cmp_ref,
                     *, H, W, H2, W2, Hs, Ws, top, left, Ch, Co, Mh, Mo):
    ktot = 2 * Ch
    offs = tuple(dy * Ws + dx for dy in range(3) for dx in range(3))

    x1t = jnp.transpose(x1_ref[0].astype(jnp.bfloat16), (1, 0))
    y = jnp.dot(x1t, wup_ref[...], preferred_element_type=jnp.float32)
    y = (y + bu_ref[...]).astype(jnp.bfloat16)

    win_ref[...] = jnp.zeros(win_ref.shape, win_ref.dtype)
    for h in range(H):
        for a in range(2):
            src = jnp.reshape(y[h * W:(h + 1) * W, a * 2 * Ch:(a + 1) * 2 * Ch],
                              (2 * W, Ch))
            base = (2 * h + a + top) * Ws + left
            win_ref[base:base + 2 * W, 0:Ch] = src
    x2t = jnp.transpose(x2_ref[0].astype(jnp.bfloat16), (1, 0))
    for r in range(H2):
        base = (r + 2) * Ws + 2
        win_ref[base:base + W2, Ch:ktot] = x2t[r * W2:(r + 1) * W2, :]

    acc = jnp.dot(win_ref[offs[0]:offs[0] + Mh, :], w1_ref[0:ktot, :],
                  preferred_element_type=jnp.float32)
    for t in range(1, 9):
        acc = acc + jnp.dot(win_ref[offs[t]:offs[t] + Mh, :],
                            w1_ref[t * ktot:(t + 1) * ktot, :],
                            preferred_element_type=jnp.float32)
    idx = jax.lax.broadcasted_iota(jnp.int32, (Mh, 1), 0)
    yy = idx // Ws
    xx = idx - yy * Ws
    keep = (yy >= 1) & (yy <= H2) & (xx >= 1) & (xx <= W2)
    h1 = jnp.maximum(acc + s1_ref[...], 0.0)
    h_ref[0:Mh, :] = jnp.where(keep, h1, 0.0).astype(h_ref.dtype)
    h_ref[Mh:, :] = jnp.zeros((h_ref.shape[0] - Mh, h_ref.shape[1]), h_ref.dtype)

    acc2 = jnp.dot(h_ref[offs[0]:offs[0] + Mo, :], w2_ref[0:Co, :],
                   preferred_element_type=jnp.float32)
    for t in range(1, 9):
        acc2 = acc2 + jnp.dot(h_ref[offs[t]:offs[t] + Mo, :],
                              w2_ref[t * Co:(t + 1) * Co, :],
                              preferred_element_type=jnp.float32)
    o2 = jnp.maximum(acc2 + s2_ref[...], 0.0)

    for r in range(H2):
        cmp_ref[r * W2:(r + 1) * W2, :] = o2[r * Ws:r * Ws + W2, :]
    o_ref[0, :, :] = jnp.transpose(cmp_ref[...], (1, 0))


def kernel(x1_nchw, x2_nchw, w_up, b_up, w1, b1, g1, be1, w2, b2, g2, be2):
    N, C1, H, W = x1_nchw.shape
    _, Ch, H2, W2 = x2_nchw.shape
    Co = int(w1.shape[0])
    Hs = H2 + 4
    Ws = _rup(W2 + 4, 8)
    top = 2 + (H2 - 2 * H) // 2
    left = 2 + (W2 - 2 * W) // 2
    ktot = 2 * Ch

    Mo = _rup((H2 - 1) * Ws + W2 + 1, 8)
    Mh = _rup(Mo + 2 * Ws + 2, 8)
    win_rows = _rup(Mh + 2 * Ws + 2, 8)

    wt = jnp.transpose(w_up, (0, 2, 3, 1)).reshape(C1, 4 * Ch).astype(jnp.bfloat16)
    bu = jnp.broadcast_to(b_up[None, None, :], (2, 2, Ch)).reshape(1, 4 * Ch)

    scale1 = g1 / jnp.sqrt(1.0 + _BN_EPS)
    w1_9 = (jnp.transpose(w1, (2, 3, 1, 0)) * scale1).reshape(9 * C1, Co)
    w1_9 = w1_9.astype(jnp.bfloat16)
    s1 = (b1 * scale1 + be1).reshape(1, Co)
    scale2 = g2 / jnp.sqrt(1.0 + _BN_EPS)
    w2_9 = (jnp.transpose(w2, (2, 3, 1, 0)) * scale2).reshape(9 * Co, Co)
    w2_9 = w2_9.astype(jnp.bfloat16)
    s2 = (b2 * scale2 + be2).reshape(1, Co)

    body = functools.partial(_up_block_kernel, H=H, W=W, H2=H2, W2=W2,
                             Hs=Hs, Ws=Ws, top=top, left=left, Ch=Ch, Co=Co,
                             Mh=Mh, Mo=Mo)
    out = pl.pallas_call(
        body,
        out_shape=jax.ShapeDtypeStruct((N, Co, H2 * W2), jnp.float32),
        grid=(N,),
        in_specs=[
            pl.BlockSpec((1, C1, H * W), lambda n: (n, 0, 0)),
            pl.BlockSpec((1, Ch, H2 * W2), lambda n: (n, 0, 0)),
            pl.BlockSpec((C1, 4 * Ch), lambda n: (0, 0)),
            pl.BlockSpec((1, 4 * Ch), lambda n: (0, 0)),
            pl.BlockSpec((9 * ktot, Co), lambda n: (0, 0)),
            pl.BlockSpec((1, Co), lambda n: (0, 0)),
            pl.BlockSpec((9 * Co, Co), lambda n: (0, 0)),
            pl.BlockSpec((1, Co), lambda n: (0, 0)),
        ],
        out_specs=pl.BlockSpec((1, Co, H2 * W2), lambda n: (n, 0, 0)),
        scratch_shapes=[
            pltpu.VMEM((win_rows, ktot), jnp.bfloat16),
            pltpu.VMEM((_rup(Mo + 2 * Ws + 2 + 8, 8), Co), jnp.bfloat16),
            pltpu.VMEM((H2 * W2, Co), jnp.float32),
        ],
        compiler_params=pltpu.CompilerParams(
            dimension_semantics=("parallel",),
            vmem_limit_bytes=64 * 1024 * 1024),
    )(x1_nchw.reshape(N, C1, H * W), x2_nchw.reshape(N, Ch, H2 * W2),
      wt, bu, w1_9, s1, w2_9, s2)
    return out.reshape(N, Co, H2, W2)


# speedup vs baseline: 1.9298x; 1.3524x over previous
import functools

import jax
import jax.numpy as jnp
from jax.experimental import pallas as pl
from jax.experimental.pallas import tpu as pltpu

_BN_EPS = 1e-5


def _rup(x, m):
    return ((x + m - 1) // m) * m


def _up_block_kernel(x1_ref, x2_ref, wup_ref, bu_ref, w1p_ref, w1d2_ref,
                     s1_ref, w2p_ref, w2d2_ref, s2_ref, o_ref,
                     win_ref, h3_ref, msk_ref, cmp_ref,
                     *, H, W, H2, W2, Ws, top, left, Ch, Co, Mc):
    ktot = 2 * Ch

    @pl.when(pl.program_id(0) == 0)
    def _init():
        win_ref[...] = jnp.zeros(win_ref.shape, win_ref.dtype)
        h3_ref[...] = jnp.zeros(h3_ref.shape, h3_ref.dtype)
        idx = jax.lax.broadcasted_iota(jnp.int32, (Mc, 1), 0) + Ws
        yy = idx // Ws
        xx = idx - yy * Ws
        keep = (xx >= 1) & (xx <= W2) & (yy <= H2)
        msk_ref[...] = jnp.where(jnp.broadcast_to(keep, (Mc, Co)),
                                 1.0, 0.0).astype(msk_ref.dtype)

    x1t = jnp.transpose(x1_ref[0].astype(jnp.bfloat16), (1, 0))
    y = jnp.dot(x1t, wup_ref[...], preferred_element_type=jnp.float32)
    y = (y + bu_ref[...]).astype(jnp.bfloat16)

    for h in range(H):
        for a in range(2):
            src = jnp.reshape(y[h * W:(h + 1) * W, a * 2 * Ch:(a + 1) * 2 * Ch],
                              (2 * W, Ch))
            base = (2 * h + a + top) * Ws + left
            for dx in range(3):
                win_ref[base - dx:base - dx + 2 * W,
                        dx * ktot:dx * ktot + Ch] = src
    x2t = jnp.transpose(x2_ref[0].astype(jnp.bfloat16), (1, 0))
    for r in range(H2):
        src = x2t[r * W2:(r + 1) * W2, :]
        base = (r + 2) * Ws + 2
        for dx in range(3):
            win_ref[base - dx:base - dx + W2,
                    dx * ktot + Ch:(dx + 1) * ktot] = src

    R = jnp.dot(win_ref[Ws:Ws + Mc + Ws, :], w1p_ref[...],
                preferred_element_type=jnp.float32)
    acc = (R[0:Mc, 0:Co] + R[Ws:Mc + Ws, Co:2 * Co]
           + jnp.dot(win_ref[3 * Ws:3 * Ws + Mc, :], w1d2_ref[...],
                     preferred_element_type=jnp.float32))
    hb = jnp.maximum(acc + s1_ref[...], 0.0).astype(jnp.bfloat16) * msk_ref[...]
    h3_ref[Ws:Ws + Mc, 0:Co] = hb
    h3_ref[Ws - 1:Ws - 1 + Mc, Co:2 * Co] = hb
    h3_ref[Ws - 2:Ws - 2 + Mc, 2 * Co:3 * Co] = hb

    R2 = jnp.dot(h3_ref[0:Mc + Ws, :], w2p_ref[...],
                 preferred_element_type=jnp.float32)
    acc2 = (R2[0:Mc, 0:Co] + R2[Ws:Mc + Ws, Co:2 * Co]
            + jnp.dot(h3_ref[2 * Ws:2 * Ws + Mc, :], w2d2_ref[...],
                      preferred_element_type=jnp.float32))
    o2 = jnp.maximum(acc2 + s2_ref[...], 0.0)

    for r in range(H2):
        cmp_ref[r * W2:(r + 1) * W2, :] = o2[r * Ws:r * Ws + W2, :]
    o_ref[0, :, :] = jnp.transpose(cmp_ref[...], (1, 0))


def kernel(x1_nchw, x2_nchw, w_up, b_up, w1, b1, g1, be1, w2, b2, g2, be2):
    N, C1, H, W = x1_nchw.shape
    _, Ch, H2, W2 = x2_nchw.shape
    Co = int(w1.shape[0])
    Ws = _rup(W2 + 4, 16)
    top = 2 + (H2 - 2 * H) // 2
    left = 2 + (W2 - 2 * W) // 2
    ktot = 2 * Ch

    Mc = _rup((H2 - 1) * Ws + W2 + 2, 16)
    win_rows = 3 * Ws + Mc
    h3_rows = 2 * Ws + Mc + Ws

    wt = jnp.transpose(w_up, (0, 2, 3, 1)).reshape(C1, 4 * Ch).astype(jnp.bfloat16)
    bu = jnp.broadcast_to(b_up[None, None, :], (2, 2, Ch)).reshape(1, 4 * Ch)

    scale1 = g1 / jnp.sqrt(1.0 + _BN_EPS)
    w1t = (jnp.transpose(w1, (2, 3, 1, 0)) * scale1).astype(jnp.bfloat16)
    w1p = jnp.concatenate([w1t[0].reshape(3 * ktot, Co),
                           w1t[1].reshape(3 * ktot, Co)], axis=1)
    w1d2 = w1t[2].reshape(3 * ktot, Co)
    s1 = (b1 * scale1 + be1).reshape(1, Co)
    scale2 = g2 / jnp.sqrt(1.0 + _BN_EPS)
    w2t = (jnp.transpose(w2, (2, 3, 1, 0)) * scale2).astype(jnp.bfloat16)
    w2p = jnp.concatenate([w2t[0].reshape(3 * Co, Co),
                           w2t[1].reshape(3 * Co, Co)], axis=1)
    w2d2 = w2t[2].reshape(3 * Co, Co)
    s2 = (b2 * scale2 + be2).reshape(1, Co)

    body = functools.partial(_up_block_kernel, H=H, W=W, H2=H2, W2=W2,
                             Ws=Ws, top=top, left=left, Ch=Ch, Co=Co, Mc=Mc)
    out = pl.pallas_call(
        body,
        out_shape=jax.ShapeDtypeStruct((N, Co, H2 * W2), jnp.float32),
        grid=(N,),
        in_specs=[
            pl.BlockSpec((1, C1, H * W), lambda n: (n, 0, 0)),
            pl.BlockSpec((1, Ch, H2 * W2), lambda n: (n, 0, 0)),
            pl.BlockSpec((C1, 4 * Ch), lambda n: (0, 0)),
            pl.BlockSpec((1, 4 * Ch), lambda n: (0, 0)),
            pl.BlockSpec((3 * ktot, 2 * Co), lambda n: (0, 0)),
            pl.BlockSpec((3 * ktot, Co), lambda n: (0, 0)),
            pl.BlockSpec((1, Co), lambda n: (0, 0)),
            pl.BlockSpec((3 * Co, 2 * Co), lambda n: (0, 0)),
            pl.BlockSpec((3 * Co, Co), lambda n: (0, 0)),
            pl.BlockSpec((1, Co), lambda n: (0, 0)),
        ],
        out_specs=pl.BlockSpec((1, Co, H2 * W2), lambda n: (n, 0, 0)),
        scratch_shapes=[
            pltpu.VMEM((win_rows, 3 * ktot), jnp.bfloat16),
            pltpu.VMEM((h3_rows, 3 * Co), jnp.bfloat16),
            pltpu.VMEM((Mc, Co), jnp.bfloat16),
            pltpu.VMEM((H2 * W2, Co), jnp.float32),
        ],
        compiler_params=pltpu.CompilerParams(
            dimension_semantics=("arbitrary",),
            vmem_limit_bytes=64 * 1024 * 1024),
    )(x1_nchw.reshape(N, C1, H * W), x2_nchw.reshape(N, Ch, H2 * W2),
      wt, bu, w1p, w1d2, s1, w2p, w2d2, s2)
    return out.reshape(N, Co, H2, W2)
